# baseline (speedup 1.0000x reference)
import functools

import jax
import jax.numpy as jnp
from jax.experimental import pallas as pl
from jax.experimental.pallas import tpu as pltpu

_G = 8


def _rup(x, m):
    return (x + m - 1) // m * m


def _pool_cols(z):
    c, n = z.shape
    if n <= 128:
        idx = jnp.broadcast_to(
            jax.lax.iota(jnp.int32, n // 2)[None, :] * 2, (c, n // 2))
        return (jnp.take_along_axis(z, idx, axis=1)
                + jnp.take_along_axis(z, idx + 1, axis=1))
    idx = jnp.broadcast_to(jax.lax.iota(jnp.int32, 64)[None, :] * 2, (c, 64))
    pieces = []
    for k in range(n // 128):
        blk = jax.lax.slice(z, (0, 128 * k), (c, 128 * (k + 1)))
        pieces.append(jnp.take_along_axis(blk, idx, axis=1)
                      + jnp.take_along_axis(blk, idx + 1, axis=1))
    return jnp.concatenate(pieces, axis=1)


def _shift1(z):
    return jnp.concatenate([z[:, 1:], z[:, :1]], axis=1)


def _succ_lanes(z, d):
    if d == 0:
        return _shift1(z)
    h = z.shape[1] // 2
    return jnp.concatenate([z[:, h:], _succ_lanes(z[:, :h], d - 1)], axis=1)


def _conv_block_poly(src_ref, out_ref, w_ref, b_ref, s_ref, t_ref,
                     d_out, bias_folded, split_sublanes):
    h_out = out_ref.shape[0]
    wv = w_ref[...]
    cout = wv.shape[0]
    if split_sublanes:
        nn = src_ref.shape[2]
    else:
        nn = src_ref.shape[2] // 2
    if not bias_folded:
        bias = jnp.broadcast_to(b_ref[...], (cout, nn))
    scale = jnp.broadcast_to(s_ref[...], (cout, nn))
    shift = jnp.broadcast_to(t_ref[...], (cout, nn))
    def _sp_row(p):
        c2 = p.shape[0] // 2
        return jnp.concatenate([p[c2:], _succ_lanes(p[:c2], 2)], axis=0)

    p0 = src_ref[0]
    sp0 = _sp_row(p0) if split_sublanes else None
    for hp in range(h_out):
        p1 = src_ref[2 * hp + 1]
        p2 = src_ref[2 * hp + 2]
        if split_sublanes:
            sp1 = _sp_row(p1)
            sp2 = _sp_row(p2)
            qe0 = jnp.concatenate([p0, p1], axis=0)
            qo0 = jnp.concatenate([sp0, sp1], axis=0)
            qe1 = jnp.concatenate([p1, p2], axis=0)
            qo1 = jnp.concatenate([sp1, sp2], axis=0)
        else:
            c = p0.shape[0] // 2
            P0, S0 = p0[:c], p0[c:]
            P1, S1 = p1[:c], p1[c:]
            P2, S2 = p2[:c], p2[c:]

            def q4(a, b):
                return jnp.concatenate(
                    [a[:, :nn], a[:, nn:], b[:, :nn], b[:, nn:]], axis=0)
            qe0, qo0 = q4(P0, P1), q4(S0, S1)
            qe1, qo1 = q4(P1, P2), q4(S1, S2)
        ys = []
        for q in (qe0, qo0, qe1, qo1):
            y = jnp.dot(wv, q, preferred_element_type=jnp.float32)
            if not bias_folded:
                y = y + bias
            ys.append(jnp.maximum(y, 0.0))
        z = (ys[0] + ys[1] + (ys[2] + ys[3])) * scale + shift
        out_ref[hp] = jnp.concatenate(
            [z, _succ_lanes(z, d_out)], axis=0).astype(jnp.bfloat16)
        p0 = p2
        if split_sublanes:
            sp0 = sp2


def _conv_block(src_ref, out_ref, w_ref, b_ref, s_ref, t_ref, store_shift,
                bias_folded=False, pmat_ref=None):
    h_out = out_ref.shape[0]
    n = src_ref.shape[2]
    wv = w_ref[...]
    cout = wv.shape[0]
    if not bias_folded:
        bias = jnp.broadcast_to(b_ref[...], (cout, n))
    scale = jnp.broadcast_to(s_ref[...], (cout, n // 2))
    shift = jnp.broadcast_to(t_ref[...], (cout, n // 2))
    pmat = pmat_ref[...] if pmat_ref is not None else None
    p0 = src_ref[0]
    for hp in range(h_out):
        p1 = src_ref[2 * hp + 1]
        p2 = src_ref[2 * hp + 2]
        t0 = jnp.concatenate([p0, p1], axis=0)
        t1 = jnp.concatenate([p1, p2], axis=0)
        y0 = jnp.dot(wv, t0, preferred_element_type=jnp.float32)
        y1 = jnp.dot(wv, t1, preferred_element_type=jnp.float32)
        if not bias_folded:
            y0 = y0 + bias
            y1 = y1 + bias
        if pmat is not None:
            z16 = (jnp.maximum(y0.astype(jnp.bfloat16), 0)
                   + jnp.maximum(y1.astype(jnp.bfloat16), 0))
            pooled = jnp.dot(z16, pmat, preferred_element_type=jnp.float32)
        else:
            pooled = _pool_cols(jnp.maximum(y0, 0.0) + jnp.maximum(y1, 0.0))
        z = pooled * scale + shift
        if store_shift:
            out_ref[hp] = jnp.concatenate([z, _shift1(z)], axis=0).astype(
                jnp.bfloat16)
        else:
            out_ref[hp] = z.astype(jnp.bfloat16)
        p0 = p2


def _enc_kernel(*args, n_conv, n_pw):
    x_ref = args[0]
    conv_refs = args[1:1 + 4 * n_conv]
    pw_refs = args[1 + 4 * n_conv:1 + 4 * (n_conv + n_pw)]
    o_ref = args[1 + 4 * (n_conv + n_pw)]
    bufs = args[2 + 4 * (n_conv + n_pw):]

    src = x_ref
    for k in range(n_conv):
        w_ref, b_ref, s_ref, t_ref = conv_refs[4 * k:4 * k + 4]
        if k <= 2:
            _conv_block_poly(src, bufs[k], w_ref, b_ref, s_ref, t_ref,
                             d_out=2 - k, bias_folded=(k <= 1),
                             split_sublanes=(k == 0))
        else:
            _conv_block(src, bufs[k], w_ref, b_ref, s_ref, t_ref,
                        store_shift=(k < n_conv - 1))
        src = bufs[k]

    v = src[0]
    u = None
    for k in range(n_pw):
        w_ref, b_ref, s_ref, t_ref = pw_refs[4 * k:4 * k + 4]
        cout = w_ref.shape[0]
        lanes = v.shape[1]
        u = jnp.maximum(
            jnp.dot(w_ref[...], v, preferred_element_type=jnp.float32)
            + jnp.broadcast_to(b_ref[...], (cout, lanes)), 0.0)
        u = u * jnp.broadcast_to(s_ref[...], (cout, lanes)) + jnp.broadcast_to(
            t_ref[...], (cout, lanes))
        v = u.astype(jnp.bfloat16)

    c_fin, lanes = u.shape
    g = lanes // 2
    sel = jnp.broadcast_to(
        jax.lax.iota(jnp.int32, g)[None, :] * 2, (c_fin, g))
    o_ref[...] = jnp.take_along_axis(u, sel, axis=1)


def _pack_conv_w(w, cin_store, eps, b, gm, bt, mn, vr, bias_col=None):
    kh, kw, cin, cout = w.shape
    cout_p = _rup(cout, 8)
    wt = jnp.zeros((cout_p, 4 * cin_store), jnp.float32)
    order = ((0, 0), (0, 1), (1, 0), (1, 1))
    blocks = {(0, 0): 0, (0, 1): 1, (1, 0): 2, (1, 1): 3}
    for (dy, dx) in order:
        blk = blocks[(dy, dx)]
        wt = wt.at[:cout, blk * cin_store:blk * cin_store + cin].set(
            w[dy, dx].astype(jnp.float32).T)
    if bias_col is not None:
        wt = wt.at[:cout, bias_col].set(b)
    s_bn = gm / jnp.sqrt(vr + eps)
    bias = jnp.zeros((cout_p, 1), jnp.float32).at[:cout, 0].set(b)
    scale = jnp.zeros((cout_p, 1), jnp.float32).at[:cout, 0].set(0.25 * s_bn)
    shift = jnp.zeros((cout_p, 1), jnp.float32).at[:cout, 0].set(
        bt - mn * s_bn)
    return wt.astype(jnp.bfloat16), bias, scale, shift


def _pack_pw(w, eps, b, gm, bt, mn, vr):
    cin, cout = w.shape
    cin_p, cout_p = _rup(cin, 8), _rup(cout, 8)
    wt = jnp.zeros((cout_p, cin_p), jnp.float32).at[:cout, :cin].set(
        w.astype(jnp.float32).T)
    s_bn = gm / jnp.sqrt(vr + eps)
    bias = jnp.zeros((cout_p, 1), jnp.float32).at[:cout, 0].set(b)
    scale = jnp.zeros((cout_p, 1), jnp.float32).at[:cout, 0].set(s_bn)
    shift = jnp.zeros((cout_p, 1), jnp.float32).at[:cout, 0].set(
        bt - mn * s_bn)
    return wt.astype(jnp.bfloat16), bias, scale, shift


def _encode(x_nchw, conv_params, pw_params, eps=1e-5):
    B, cin, H, W = x_nchw.shape
    g = _G
    ng = B // g
    cs0 = 4

    xb = x_nchw.astype(jnp.bfloat16)
    ones = jnp.ones((B, cs0 - cin, H, W), jnp.bfloat16)
    xp = jnp.concatenate([xb, ones], axis=1)
    xp = xp.reshape(ng, g, cs0, H, W).transpose(0, 3, 2, 1, 4)
    xp = xp.reshape(ng, H, cs0, g * W)
    hw = g * W // 2
    perm2 = sorted(range(hw),
                   key=lambda p: ((p % (W // 2)) & 1, (p % (W // 2) >> 1) & 1,
                                  p // (W // 2), p % (W // 2) >> 2))
    x32 = jax.lax.bitcast_convert_type(
        xp.reshape(ng, H, cs0, hw, 2), jnp.uint32)
    x32 = jnp.take(x32, jnp.asarray(perm2, jnp.int32), axis=3)
    lo = jax.lax.bitcast_convert_type(
        (x32 & 0xFFFF).astype(jnp.uint16), jnp.bfloat16)
    hi = jax.lax.bitcast_convert_type(
        (x32 >> 16).astype(jnp.uint16), jnp.bfloat16)
    xp = jnp.concatenate([lo, hi], axis=2)

    weight_arrays = []
    h, w_, cin_store = H, W, cs0
    scratch = []
    n_conv = len(conv_params)
    for k, (wt, b, gm, bt, mn, vr) in enumerate(conv_params):
        bias_col = {0: cin, 1: 2 * cs0 - 1}.get(k)
        packed = _pack_conv_w(wt, cin_store, eps, b, gm, bt, mn, vr,
                              bias_col=bias_col)
        wt_p, b_p, s_p, t_p = packed
        if k == 0:
            s_p = s_p.at[2 * cs0 - 1, 0].set(0.0)
            t_p = t_p.at[2 * cs0 - 1, 0].set(1.0)
        weight_arrays += [wt_p, b_p, s_p, t_p]
        cout_p = wt_p.shape[0]
        h, w_ = (h - 1) // 2, w_ // 2
        store_c = cout_p if k == n_conv - 1 else 2 * cout_p
        scratch.append(pltpu.VMEM((h, store_c, g * w_), jnp.bfloat16))
        cin_store = cout_p

    c_fin = cin_store
    for (wt, b, gm, bt, mn, vr) in pw_params:
        packed = _pack_pw(wt, eps, b, gm, bt, mn, vr)
        weight_arrays += list(packed)
        c_fin = packed[0].shape[0]

    def _const_spec(arr):
        nd = arr.ndim
        return pl.BlockSpec(arr.shape, lambda bi, _nd=nd: (0,) * _nd)

    in_specs = [pl.BlockSpec((None, H, 2 * cs0, g * W // 2),
                             lambda bi: (bi, 0, 0, 0))]
    in_specs += [_const_spec(a) for a in weight_arrays]

    out = pl.pallas_call(
        functools.partial(_enc_kernel, n_conv=n_conv, n_pw=len(pw_params)),
        out_shape=jax.ShapeDtypeStruct((ng, c_fin, g), jnp.float32),
        grid=(ng,),
        in_specs=in_specs,
        out_specs=pl.BlockSpec((None, c_fin, g), lambda bi: (bi, 0, 0)),
        scratch_shapes=scratch,
        compiler_params=pltpu.CompilerParams(
            dimension_semantics=("parallel",)),
    )(xp, *weight_arrays)
    return out.transpose(0, 2, 1).reshape(B, c_fin, 1, 1)


def kernel(
    x_nchw,
    conv0_w, conv0_b, conv0_g, conv0_be, conv0_m, conv0_v,
    conv1_w, conv1_b, conv1_g, conv1_be, conv1_m, conv1_v,
    conv2_w, conv2_b, conv2_g, conv2_be, conv2_m, conv2_v,
    conv3_w, conv3_b, conv3_g, conv3_be, conv3_m, conv3_v,
    conv4_w, conv4_b, conv4_g, conv4_be, conv4_m, conv4_v,
    conv5_w, conv5_b, conv5_g, conv5_be, conv5_m, conv5_v,
    pw0_w, pw0_b, pw0_g, pw0_be, pw0_m, pw0_v,
    pw1_w, pw1_b, pw1_g, pw1_be, pw1_m, pw1_v,
):
    conv_params = [
        (conv0_w, conv0_b, conv0_g, conv0_be, conv0_m, conv0_v),
        (conv1_w, conv1_b, conv1_g, conv1_be, conv1_m, conv1_v),
        (conv2_w, conv2_b, conv2_g, conv2_be, conv2_m, conv2_v),
        (conv3_w, conv3_b, conv3_g, conv3_be, conv3_m, conv3_v),
        (conv4_w, conv4_b, conv4_g, conv4_be, conv4_m, conv4_v),
        (conv5_w, conv5_b, conv5_g, conv5_be, conv5_m, conv5_v),
    ]
    pw_params = [
        (pw0_w, pw0_b, pw0_g, pw0_be, pw0_m, pw0_v),
        (pw1_w, pw1_b, pw1_g, pw1_be, pw1_m, pw1_v),
    ]
    return _encode(x_nchw, conv_params, pw_params)


# speedup vs baseline: 1.0491x; 1.0491x over previous
import functools

import jax
import jax.numpy as jnp
from jax.experimental import pallas as pl
from jax.experimental.pallas import tpu as pltpu

_G = 8


def _rup(x, m):
    return (x + m - 1) // m * m


def _pool_cols(z):
    c, n = z.shape
    if n <= 128:
        idx = jnp.broadcast_to(
            jax.lax.iota(jnp.int32, n // 2)[None, :] * 2, (c, n // 2))
        return (jnp.take_along_axis(z, idx, axis=1)
                + jnp.take_along_axis(z, idx + 1, axis=1))
    idx = jnp.broadcast_to(jax.lax.iota(jnp.int32, 64)[None, :] * 2, (c, 64))
    pieces = []
    for k in range(n // 128):
        blk = jax.lax.slice(z, (0, 128 * k), (c, 128 * (k + 1)))
        pieces.append(jnp.take_along_axis(blk, idx, axis=1)
                      + jnp.take_along_axis(blk, idx + 1, axis=1))
    return jnp.concatenate(pieces, axis=1)


def _shift1(z):
    return jnp.concatenate([z[:, 1:], z[:, :1]], axis=1)


def _succ_lanes(z, d):
    if d == 0:
        return _shift1(z)
    h = z.shape[1] // 2
    return jnp.concatenate([z[:, h:], _succ_lanes(z[:, :h], d - 1)], axis=1)


def _conv_block_poly(src_ref, out_ref, w_ref, b_ref, s_ref, t_ref,
                     d_out, bias_folded, split_sublanes):
    h_out = out_ref.shape[0]
    wv = w_ref[...]
    cout = wv.shape[0]
    if split_sublanes:
        nn = src_ref.shape[2]
    else:
        nn = src_ref.shape[2] // 2
    if not bias_folded:
        bias = jnp.broadcast_to(b_ref[...], (cout, 2 * nn))
    scale = jnp.broadcast_to(s_ref[...], (cout, nn))
    shift = jnp.broadcast_to(t_ref[...], (cout, nn))
    def _row_ext(p):
        if split_sublanes:
            c2 = p.shape[0] // 2
            sp = jnp.concatenate([p[c2:], _succ_lanes(p[:c2], 2)], axis=0)
            return jnp.concatenate([p, sp], axis=1)
        c = p.shape[0] // 2
        P, S = p[:c], p[c:]
        return jnp.concatenate(
            [jnp.concatenate([P[:, :nn], S[:, :nn]], axis=1),
             jnp.concatenate([P[:, nn:], S[:, nn:]], axis=1)], axis=0)

    r0 = _row_ext(src_ref[0])
    for hp in range(h_out):
        r1 = _row_ext(src_ref[2 * hp + 1])
        r2 = _row_ext(src_ref[2 * hp + 2])
        ys = []
        for q in (jnp.concatenate([r0, r1], axis=0),
                  jnp.concatenate([r1, r2], axis=0)):
            y = jnp.dot(wv, q, preferred_element_type=jnp.float32)
            if not bias_folded:
                y = y + bias
            ys.append(jnp.maximum(y, 0.0))
        z2 = ys[0] + ys[1]
        z = (z2[:, :nn] + z2[:, nn:]) * scale + shift
        out_ref[hp] = jnp.concatenate(
            [z, _succ_lanes(z, d_out)], axis=0).astype(jnp.bfloat16)
        r0 = r2


def _conv_block(src_ref, out_ref, w_ref, b_ref, s_ref, t_ref, store_shift,
                bias_folded=False, pmat_ref=None):
    h_out = out_ref.shape[0]
    n = src_ref.shape[2]
    wv = w_ref[...]
    cout = wv.shape[0]
    if not bias_folded:
        bias = jnp.broadcast_to(b_ref[...], (cout, n))
    scale = jnp.broadcast_to(s_ref[...], (cout, n // 2))
    shift = jnp.broadcast_to(t_ref[...], (cout, n // 2))
    pmat = pmat_ref[...] if pmat_ref is not None else None
    p0 = src_ref[0]
    for hp in range(h_out):
        p1 = src_ref[2 * hp + 1]
        p2 = src_ref[2 * hp + 2]
        t0 = jnp.concatenate([p0, p1], axis=0)
        t1 = jnp.concatenate([p1, p2], axis=0)
        y0 = jnp.dot(wv, t0, preferred_element_type=jnp.float32)
        y1 = jnp.dot(wv, t1, preferred_element_type=jnp.float32)
        if not bias_folded:
            y0 = y0 + bias
            y1 = y1 + bias
        if pmat is not None:
            z16 = (jnp.maximum(y0.astype(jnp.bfloat16), 0)
                   + jnp.maximum(y1.astype(jnp.bfloat16), 0))
            pooled = jnp.dot(z16, pmat, preferred_element_type=jnp.float32)
        else:
            pooled = _pool_cols(jnp.maximum(y0, 0.0) + jnp.maximum(y1, 0.0))
        z = pooled * scale + shift
        if store_shift:
            out_ref[hp] = jnp.concatenate([z, _shift1(z)], axis=0).astype(
                jnp.bfloat16)
        else:
            out_ref[hp] = z.astype(jnp.bfloat16)
        p0 = p2


def _enc_kernel(*args, n_conv, n_pw):
    x_ref = args[0]
    conv_refs = args[1:1 + 4 * n_conv]
    pw_refs = args[1 + 4 * n_conv:1 + 4 * (n_conv + n_pw)]
    o_ref = args[1 + 4 * (n_conv + n_pw)]
    bufs = args[2 + 4 * (n_conv + n_pw):]

    src = x_ref
    for k in range(n_conv):
        w_ref, b_ref, s_ref, t_ref = conv_refs[4 * k:4 * k + 4]
        if k <= 2:
            _conv_block_poly(src, bufs[k], w_ref, b_ref, s_ref, t_ref,
                             d_out=2 - k, bias_folded=(k <= 1),
                             split_sublanes=(k == 0))
        else:
            _conv_block(src, bufs[k], w_ref, b_ref, s_ref, t_ref,
                        store_shift=(k < n_conv - 1))
        src = bufs[k]

    v = src[0]
    u = None
    for k in range(n_pw):
        w_ref, b_ref, s_ref, t_ref = pw_refs[4 * k:4 * k + 4]
        cout = w_ref.shape[0]
        lanes = v.shape[1]
        u = jnp.maximum(
            jnp.dot(w_ref[...], v, preferred_element_type=jnp.float32)
            + jnp.broadcast_to(b_ref[...], (cout, lanes)), 0.0)
        u = u * jnp.broadcast_to(s_ref[...], (cout, lanes)) + jnp.broadcast_to(
            t_ref[...], (cout, lanes))
        v = u.astype(jnp.bfloat16)

    c_fin, lanes = u.shape
    g = lanes // 2
    sel = jnp.broadcast_to(
        jax.lax.iota(jnp.int32, g)[None, :] * 2, (c_fin, g))
    o_ref[...] = jnp.take_along_axis(u, sel, axis=1)


def _pack_conv_w(w, cin_store, eps, b, gm, bt, mn, vr, bias_col=None):
    kh, kw, cin, cout = w.shape
    cout_p = _rup(cout, 8)
    wt = jnp.zeros((cout_p, 4 * cin_store), jnp.float32)
    order = ((0, 0), (0, 1), (1, 0), (1, 1))
    blocks = {(0, 0): 0, (0, 1): 1, (1, 0): 2, (1, 1): 3}
    for (dy, dx) in order:
        blk = blocks[(dy, dx)]
        wt = wt.at[:cout, blk * cin_store:blk * cin_store + cin].set(
            w[dy, dx].astype(jnp.float32).T)
    if bias_col is not None:
        wt = wt.at[:cout, bias_col].set(b)
    s_bn = gm / jnp.sqrt(vr + eps)
    bias = jnp.zeros((cout_p, 1), jnp.float32).at[:cout, 0].set(b)
    scale = jnp.zeros((cout_p, 1), jnp.float32).at[:cout, 0].set(0.25 * s_bn)
    shift = jnp.zeros((cout_p, 1), jnp.float32).at[:cout, 0].set(
        bt - mn * s_bn)
    return wt.astype(jnp.bfloat16), bias, scale, shift


def _pack_pw(w, eps, b, gm, bt, mn, vr):
    cin, cout = w.shape
    cin_p, cout_p = _rup(cin, 8), _rup(cout, 8)
    wt = jnp.zeros((cout_p, cin_p), jnp.float32).at[:cout, :cin].set(
        w.astype(jnp.float32).T)
    s_bn = gm / jnp.sqrt(vr + eps)
    bias = jnp.zeros((cout_p, 1), jnp.float32).at[:cout, 0].set(b)
    scale = jnp.zeros((cout_p, 1), jnp.float32).at[:cout, 0].set(s_bn)
    shift = jnp.zeros((cout_p, 1), jnp.float32).at[:cout, 0].set(
        bt - mn * s_bn)
    return wt.astype(jnp.bfloat16), bias, scale, shift


def _encode(x_nchw, conv_params, pw_params, eps=1e-5):
    B, cin, H, W = x_nchw.shape
    g = _G
    ng = B // g
    cs0 = 4

    xb = x_nchw.astype(jnp.bfloat16)
    ones = jnp.ones((B, cs0 - cin, H, W), jnp.bfloat16)
    xp = jnp.concatenate([xb, ones], axis=1)
    xp = xp.reshape(ng, g, cs0, H, W).transpose(0, 3, 2, 1, 4)
    xp = xp.reshape(ng, H, cs0, g * W)
    perm = sorted(range(g * W),
                  key=lambda p: ((p % W) & 1, (p % W >> 1) & 1,
                                 (p % W >> 2) & 1, p // W, p % W >> 3))
    hw = g * W // 2
    xa = jnp.take(xp, jnp.asarray(perm, jnp.int32), axis=3)
    xp = jnp.concatenate([xa[..., :hw], xa[..., hw:]], axis=2)

    weight_arrays = []
    h, w_, cin_store = H, W, cs0
    scratch = []
    n_conv = len(conv_params)
    for k, (wt, b, gm, bt, mn, vr) in enumerate(conv_params):
        bias_col = {0: cin, 1: 2 * cs0 - 1}.get(k)
        packed = _pack_conv_w(wt, cin_store, eps, b, gm, bt, mn, vr,
                              bias_col=bias_col)
        wt_p, b_p, s_p, t_p = packed
        if k == 0:
            s_p = s_p.at[2 * cs0 - 1, 0].set(0.0)
            t_p = t_p.at[2 * cs0 - 1, 0].set(1.0)
        weight_arrays += [wt_p, b_p, s_p, t_p]
        cout_p = wt_p.shape[0]
        h, w_ = (h - 1) // 2, w_ // 2
        store_c = cout_p if k == n_conv - 1 else 2 * cout_p
        scratch.append(pltpu.VMEM((h, store_c, g * w_), jnp.bfloat16))
        cin_store = cout_p

    c_fin = cin_store
    for (wt, b, gm, bt, mn, vr) in pw_params:
        packed = _pack_pw(wt, eps, b, gm, bt, mn, vr)
        weight_arrays += list(packed)
        c_fin = packed[0].shape[0]

    def _const_spec(arr):
        nd = arr.ndim
        return pl.BlockSpec(arr.shape, lambda bi, _nd=nd: (0,) * _nd)

    in_specs = [pl.BlockSpec((None, H, 2 * cs0, g * W // 2),
                             lambda bi: (bi, 0, 0, 0))]
    in_specs += [_const_spec(a) for a in weight_arrays]

    out = pl.pallas_call(
        functools.partial(_enc_kernel, n_conv=n_conv, n_pw=len(pw_params)),
        out_shape=jax.ShapeDtypeStruct((ng, c_fin, g), jnp.float32),
        grid=(ng,),
        in_specs=in_specs,
        out_specs=pl.BlockSpec((None, c_fin, g), lambda bi: (bi, 0, 0)),
        scratch_shapes=scratch,
        compiler_params=pltpu.CompilerParams(
            dimension_semantics=("parallel",)),
    )(xp, *weight_arrays)
    return out.transpose(0, 2, 1).reshape(B, c_fin, 1, 1)


def kernel(
    x_nchw,
    conv0_w, conv0_b, conv0_g, conv0_be, conv0_m, conv0_v,
    conv1_w, conv1_b, conv1_g, conv1_be, conv1_m, conv1_v,
    conv2_w, conv2_b, conv2_g, conv2_be, conv2_m, conv2_v,
    conv3_w, conv3_b, conv3_g, conv3_be, conv3_m, conv3_v,
    conv4_w, conv4_b, conv4_g, conv4_be, conv4_m, conv4_v,
    conv5_w, conv5_b, conv5_g, conv5_be, conv5_m, conv5_v,
    pw0_w, pw0_b, pw0_g, pw0_be, pw0_m, pw0_v,
    pw1_w, pw1_b, pw1_g, pw1_be, pw1_m, pw1_v,
):
    conv_params = [
        (conv0_w, conv0_b, conv0_g, conv0_be, conv0_m, conv0_v),
        (conv1_w, conv1_b, conv1_g, conv1_be, conv1_m, conv1_v),
        (conv2_w, conv2_b, conv2_g, conv2_be, conv2_m, conv2_v),
        (conv3_w, conv3_b, conv3_g, conv3_be, conv3_m, conv3_v),
        (conv4_w, conv4_b, conv4_g, conv4_be, conv4_m, conv4_v),
        (conv5_w, conv5_b, conv5_g, conv5_be, conv5_m, conv5_v),
    ]
    pw_params = [
        (pw0_w, pw0_b, pw0_g, pw0_be, pw0_m, pw0_v),
        (pw1_w, pw1_b, pw1_g, pw1_be, pw1_m, pw1_v),
    ]
    return _encode(x_nchw, conv_params, pw_params)


# speedup vs baseline: 1.0729x; 1.0227x over previous
import functools

import jax
import jax.numpy as jnp
from jax.experimental import pallas as pl
from jax.experimental.pallas import tpu as pltpu

_G = 8


def _rup(x, m):
    return (x + m - 1) // m * m


def _pool_cols(z):
    c, n = z.shape
    if n <= 128:
        idx = jnp.broadcast_to(
            jax.lax.iota(jnp.int32, n // 2)[None, :] * 2, (c, n // 2))
        return (jnp.take_along_axis(z, idx, axis=1)
                + jnp.take_along_axis(z, idx + 1, axis=1))
    idx = jnp.broadcast_to(jax.lax.iota(jnp.int32, 64)[None, :] * 2, (c, 64))
    pieces = []
    for k in range(n // 128):
        blk = jax.lax.slice(z, (0, 128 * k), (c, 128 * (k + 1)))
        pieces.append(jnp.take_along_axis(blk, idx, axis=1)
                      + jnp.take_along_axis(blk, idx + 1, axis=1))
    return jnp.concatenate(pieces, axis=1)


def _shift1(z):
    return jnp.concatenate([z[:, 1:], z[:, :1]], axis=1)


def _succ_lanes(z, d):
    if d == 0:
        return _shift1(z)
    h = z.shape[1] // 2
    return jnp.concatenate([z[:, h:], _succ_lanes(z[:, :h], d - 1)], axis=1)


def _conv_block_poly(src_ref, out_ref, w_ref, b_ref, s_ref, t_ref,
                     d_out, bias_folded, split_sublanes):
    h_out = out_ref.shape[0]
    wv = w_ref[...]
    cout = wv.shape[0]
    if split_sublanes:
        nn = src_ref.shape[2]
    else:
        nn = src_ref.shape[2] // 2
    if not bias_folded:
        bias = jnp.broadcast_to(b_ref[...], (cout, nn))
    scale = jnp.broadcast_to(s_ref[...], (cout, nn))
    shift = jnp.broadcast_to(t_ref[...], (cout, nn))
    def _sp_row(p):
        c2 = p.shape[0] // 2
        return jnp.concatenate([p[c2:], _succ_lanes(p[:c2], 2)], axis=0)

    p0 = src_ref[0]
    sp0 = _sp_row(p0) if split_sublanes else None
    for hp in range(h_out):
        p1 = src_ref[2 * hp + 1]
        p2 = src_ref[2 * hp + 2]
        if split_sublanes:
            sp1 = _sp_row(p1)
            sp2 = _sp_row(p2)
            qe0 = jnp.concatenate([p0, p1], axis=0)
            qo0 = jnp.concatenate([sp0, sp1], axis=0)
            qe1 = jnp.concatenate([p1, p2], axis=0)
            qo1 = jnp.concatenate([sp1, sp2], axis=0)
        else:
            c = p0.shape[0] // 2
            P0, S0 = p0[:c], p0[c:]
            P1, S1 = p1[:c], p1[c:]
            P2, S2 = p2[:c], p2[c:]

            def q4(a, b):
                return jnp.concatenate(
                    [a[:, :nn], a[:, nn:], b[:, :nn], b[:, nn:]], axis=0)
            qe0, qo0 = q4(P0, P1), q4(S0, S1)
            qe1, qo1 = q4(P1, P2), q4(S1, S2)
        ys = []
        for q in (qe0, qo0, qe1, qo1):
            y = jnp.dot(wv, q, preferred_element_type=jnp.float32)
            if not bias_folded:
                y = y + bias
            ys.append(jnp.maximum(y, 0.0))
        z = (ys[0] + ys[1] + (ys[2] + ys[3])) * scale + shift
        out_ref[hp] = jnp.concatenate(
            [z, _succ_lanes(z, d_out)], axis=0).astype(jnp.bfloat16)
        p0 = p2
        if split_sublanes:
            sp0 = sp2


def _conv_block(src_ref, out_ref, w_ref, b_ref, s_ref, t_ref, store_shift,
                bias_folded=False, pmat_ref=None):
    h_out = out_ref.shape[0]
    n = src_ref.shape[2]
    wv = w_ref[...]
    cout = wv.shape[0]
    if not bias_folded:
        bias = jnp.broadcast_to(b_ref[...], (cout, n))
    scale = jnp.broadcast_to(s_ref[...], (cout, n // 2))
    shift = jnp.broadcast_to(t_ref[...], (cout, n // 2))
    pmat = pmat_ref[...] if pmat_ref is not None else None
    p0 = src_ref[0]
    for hp in range(h_out):
        p1 = src_ref[2 * hp + 1]
        p2 = src_ref[2 * hp + 2]
        t0 = jnp.concatenate([p0, p1], axis=0)
        t1 = jnp.concatenate([p1, p2], axis=0)
        y0 = jnp.dot(wv, t0, preferred_element_type=jnp.float32)
        y1 = jnp.dot(wv, t1, preferred_element_type=jnp.float32)
        if not bias_folded:
            y0 = y0 + bias
            y1 = y1 + bias
        if pmat is not None:
            z16 = (jnp.maximum(y0.astype(jnp.bfloat16), 0)
                   + jnp.maximum(y1.astype(jnp.bfloat16), 0))
            pooled = jnp.dot(z16, pmat, preferred_element_type=jnp.float32)
        else:
            pooled = _pool_cols(jnp.maximum(y0, 0.0) + jnp.maximum(y1, 0.0))
        z = pooled * scale + shift
        if store_shift:
            out_ref[hp] = jnp.concatenate([z, _shift1(z)], axis=0).astype(
                jnp.bfloat16)
        else:
            out_ref[hp] = z.astype(jnp.bfloat16)
        p0 = p2


def _enc_kernel(*args, n_conv, n_pw):
    x_ref = args[0]
    conv_refs = args[1:1 + 4 * n_conv]
    pw_refs = args[1 + 4 * n_conv:1 + 4 * (n_conv + n_pw)]
    o_ref = args[1 + 4 * (n_conv + n_pw)]
    bufs = args[2 + 4 * (n_conv + n_pw):]

    src = x_ref
    for k in range(n_conv):
        w_ref, b_ref, s_ref, t_ref = conv_refs[4 * k:4 * k + 4]
        if k <= 2:
            _conv_block_poly(src, bufs[k], w_ref, b_ref, s_ref, t_ref,
                             d_out=2 - k, bias_folded=(k <= 1),
                             split_sublanes=(k == 0))
        else:
            _conv_block(src, bufs[k], w_ref, b_ref, s_ref, t_ref,
                        store_shift=(k < n_conv - 1))
        src = bufs[k]

    v = src[0]
    u = None
    for k in range(n_pw):
        w_ref, b_ref, s_ref, t_ref = pw_refs[4 * k:4 * k + 4]
        cout = w_ref.shape[0]
        lanes = v.shape[1]
        u = jnp.maximum(
            jnp.dot(w_ref[...], v, preferred_element_type=jnp.float32)
            + jnp.broadcast_to(b_ref[...], (cout, lanes)), 0.0)
        u = u * jnp.broadcast_to(s_ref[...], (cout, lanes)) + jnp.broadcast_to(
            t_ref[...], (cout, lanes))
        v = u.astype(jnp.bfloat16)

    c_fin, lanes = u.shape
    g = lanes // 2
    sel = jnp.broadcast_to(
        jax.lax.iota(jnp.int32, g)[None, :] * 2, (c_fin, g))
    o_ref[...] = jnp.take_along_axis(u, sel, axis=1)


def _pack_conv_w(w, cin_store, eps, b, gm, bt, mn, vr, bias_col=None):
    kh, kw, cin, cout = w.shape
    cout_p = _rup(cout, 8)
    wt = jnp.zeros((cout_p, 4 * cin_store), jnp.float32)
    order = ((0, 0), (0, 1), (1, 0), (1, 1))
    blocks = {(0, 0): 0, (0, 1): 1, (1, 0): 2, (1, 1): 3}
    for (dy, dx) in order:
        blk = blocks[(dy, dx)]
        wt = wt.at[:cout, blk * cin_store:blk * cin_store + cin].set(
            w[dy, dx].astype(jnp.float32).T)
    if bias_col is not None:
        wt = wt.at[:cout, bias_col].set(b)
    s_bn = gm / jnp.sqrt(vr + eps)
    bias = jnp.zeros((cout_p, 1), jnp.float32).at[:cout, 0].set(b)
    scale = jnp.zeros((cout_p, 1), jnp.float32).at[:cout, 0].set(0.25 * s_bn)
    shift = jnp.zeros((cout_p, 1), jnp.float32).at[:cout, 0].set(
        bt - mn * s_bn)
    return wt.astype(jnp.bfloat16), bias, scale, shift


def _pack_pw(w, eps, b, gm, bt, mn, vr):
    cin, cout = w.shape
    cin_p, cout_p = _rup(cin, 8), _rup(cout, 8)
    wt = jnp.zeros((cout_p, cin_p), jnp.float32).at[:cout, :cin].set(
        w.astype(jnp.float32).T)
    s_bn = gm / jnp.sqrt(vr + eps)
    bias = jnp.zeros((cout_p, 1), jnp.float32).at[:cout, 0].set(b)
    scale = jnp.zeros((cout_p, 1), jnp.float32).at[:cout, 0].set(s_bn)
    shift = jnp.zeros((cout_p, 1), jnp.float32).at[:cout, 0].set(
        bt - mn * s_bn)
    return wt.astype(jnp.bfloat16), bias, scale, shift


def _encode(x_nchw, conv_params, pw_params, eps=1e-5):
    B, cin, H, W = x_nchw.shape
    g = _G
    ng = B // g
    cs0 = 4

    xb = x_nchw.astype(jnp.bfloat16)
    ones = jnp.ones((B, cs0 - cin, H, W), jnp.bfloat16)
    xp = jnp.concatenate([xb, ones], axis=1)
    xp = xp.reshape(ng, g, cs0, H, W).transpose(0, 3, 2, 1, 4)
    xp = xp.reshape(ng, H, cs0, g * W)
    perm = sorted(range(g * W),
                  key=lambda p: ((p % W) & 1, (p % W >> 1) & 1,
                                 (p % W >> 2) & 1, p // W, p % W >> 3))
    hw = g * W // 2
    xa = jnp.take(xp, jnp.asarray(perm, jnp.int32), axis=3)
    xp = jnp.concatenate([xa[..., :hw], xa[..., hw:]], axis=2)

    weight_arrays = []
    h, w_, cin_store = H, W, cs0
    scratch = []
    n_conv = len(conv_params)
    for k, (wt, b, gm, bt, mn, vr) in enumerate(conv_params):
        bias_col = {0: cin, 1: 2 * cs0 - 1}.get(k)
        packed = _pack_conv_w(wt, cin_store, eps, b, gm, bt, mn, vr,
                              bias_col=bias_col)
        wt_p, b_p, s_p, t_p = packed
        if k == 0:
            s_p = s_p.at[2 * cs0 - 1, 0].set(0.0)
            t_p = t_p.at[2 * cs0 - 1, 0].set(1.0)
        weight_arrays += [wt_p, b_p, s_p, t_p]
        cout_p = wt_p.shape[0]
        h, w_ = (h - 1) // 2, w_ // 2
        store_c = cout_p if k == n_conv - 1 else 2 * cout_p
        scratch.append(pltpu.VMEM((h, store_c, g * w_), jnp.bfloat16))
        cin_store = cout_p

    c_fin = cin_store
    for (wt, b, gm, bt, mn, vr) in pw_params:
        packed = _pack_pw(wt, eps, b, gm, bt, mn, vr)
        weight_arrays += list(packed)
        c_fin = packed[0].shape[0]

    def _const_spec(arr):
        nd = arr.ndim
        return pl.BlockSpec(arr.shape, lambda bi, _nd=nd: (0,) * _nd)

    in_specs = [pl.BlockSpec((None, H, 2 * cs0, g * W // 2),
                             lambda bi: (bi, 0, 0, 0))]
    in_specs += [_const_spec(a) for a in weight_arrays]

    out = pl.pallas_call(
        functools.partial(_enc_kernel, n_conv=n_conv, n_pw=len(pw_params)),
        out_shape=jax.ShapeDtypeStruct((ng, c_fin, g), jnp.float32),
        grid=(ng,),
        in_specs=in_specs,
        out_specs=pl.BlockSpec((None, c_fin, g), lambda bi: (bi, 0, 0)),
        scratch_shapes=scratch,
        compiler_params=pltpu.CompilerParams(
            dimension_semantics=("parallel",)),
    )(xp, *weight_arrays)
    return out.transpose(0, 2, 1).reshape(B, c_fin, 1, 1)


def kernel(
    x_nchw,
    conv0_w, conv0_b, conv0_g, conv0_be, conv0_m, conv0_v,
    conv1_w, conv1_b, conv1_g, conv1_be, conv1_m, conv1_v,
    conv2_w, conv2_b, conv2_g, conv2_be, conv2_m, conv2_v,
    conv3_w, conv3_b, conv3_g, conv3_be, conv3_m, conv3_v,
    conv4_w, conv4_b, conv4_g, conv4_be, conv4_m, conv4_v,
    conv5_w, conv5_b, conv5_g, conv5_be, conv5_m, conv5_v,
    pw0_w, pw0_b, pw0_g, pw0_be, pw0_m, pw0_v,
    pw1_w, pw1_b, pw1_g, pw1_be, pw1_m, pw1_v,
):
    conv_params = [
        (conv0_w, conv0_b, conv0_g, conv0_be, conv0_m, conv0_v),
        (conv1_w, conv1_b, conv1_g, conv1_be, conv1_m, conv1_v),
        (conv2_w, conv2_b, conv2_g, conv2_be, conv2_m, conv2_v),
        (conv3_w, conv3_b, conv3_g, conv3_be, conv3_m, conv3_v),
        (conv4_w, conv4_b, conv4_g, conv4_be, conv4_m, conv4_v),
        (conv5_w, conv5_b, conv5_g, conv5_be, conv5_m, conv5_v),
    ]
    pw_params = [
        (pw0_w, pw0_b, pw0_g, pw0_be, pw0_m, pw0_v),
        (pw1_w, pw1_b, pw1_g, pw1_be, pw1_m, pw1_v),
    ]
    return _encode(x_nchw, conv_params, pw_params)


# speedup vs baseline: 1.0732x; 1.0003x over previous
import functools

import jax
import jax.numpy as jnp
from jax.experimental import pallas as pl
from jax.experimental.pallas import tpu as pltpu

_G = 8


def _rup(x, m):
    return (x + m - 1) // m * m


def _pool_cols(z):
    c, n = z.shape
    if n <= 128:
        idx = jnp.broadcast_to(
            jax.lax.iota(jnp.int32, n // 2)[None, :] * 2, (c, n // 2))
        return (jnp.take_along_axis(z, idx, axis=1)
                + jnp.take_along_axis(z, idx + 1, axis=1))
    idx = jnp.broadcast_to(jax.lax.iota(jnp.int32, 64)[None, :] * 2, (c, 64))
    pieces = []
    for k in range(n // 128):
        blk = jax.lax.slice(z, (0, 128 * k), (c, 128 * (k + 1)))
        pieces.append(jnp.take_along_axis(blk, idx, axis=1)
                      + jnp.take_along_axis(blk, idx + 1, axis=1))
    return jnp.concatenate(pieces, axis=1)


def _shift1(z):
    return jnp.concatenate([z[:, 1:], z[:, :1]], axis=1)


def _succ_lanes(z, d):
    if d == 0:
        return _shift1(z)
    h = z.shape[1] // 2
    return jnp.concatenate([z[:, h:], _succ_lanes(z[:, :h], d - 1)], axis=1)


def _conv_block_poly(src_ref, out_ref, w_ref, b_ref, s_ref, t_ref,
                     d_out, bias_folded, split_sublanes):
    h_out = out_ref.shape[0]
    wv = w_ref[...]
    cout = wv.shape[0]
    if split_sublanes:
        nn = src_ref.shape[2]
    else:
        nn = src_ref.shape[2] // 2
    if not bias_folded:
        bias = jnp.broadcast_to(b_ref[...], (cout, nn))
    scale = jnp.broadcast_to(s_ref[...], (cout, nn))
    shift = jnp.broadcast_to(t_ref[...], (cout, nn))
    def _sp_row(p):
        c2 = p.shape[0] // 2
        return jnp.concatenate([p[c2:], _succ_lanes(p[:c2], 2)], axis=0)

    p0 = src_ref[0]
    sp0 = _sp_row(p0) if split_sublanes else None
    for hp in range(h_out):
        p1 = src_ref[2 * hp + 1]
        p2 = src_ref[2 * hp + 2]
        if split_sublanes:
            sp1 = _sp_row(p1)
            sp2 = _sp_row(p2)
            qe0 = jnp.concatenate([p0, p1], axis=0)
            qo0 = jnp.concatenate([sp0, sp1], axis=0)
            qe1 = jnp.concatenate([p1, p2], axis=0)
            qo1 = jnp.concatenate([sp1, sp2], axis=0)
        else:
            c = p0.shape[0] // 2
            P0, S0 = p0[:c], p0[c:]
            P1, S1 = p1[:c], p1[c:]
            P2, S2 = p2[:c], p2[c:]

            def q4(a, b):
                return jnp.concatenate(
                    [a[:, :nn], a[:, nn:], b[:, :nn], b[:, nn:]], axis=0)
            qe0, qo0 = q4(P0, P1), q4(S0, S1)
            qe1, qo1 = q4(P1, P2), q4(S1, S2)
        ys = []
        for q in (qe0, qo0, qe1, qo1):
            y = jnp.dot(wv, q, preferred_element_type=jnp.float32)
            if not bias_folded:
                y = y + bias
            ys.append(jnp.maximum(y, 0.0))
        z = (ys[0] + ys[1] + (ys[2] + ys[3])) * scale + shift
        out_ref[hp] = jnp.concatenate(
            [z, _succ_lanes(z, d_out)], axis=0).astype(jnp.bfloat16)
        p0 = p2
        if split_sublanes:
            sp0 = sp2


def _conv_block(src_ref, out_ref, w_ref, b_ref, s_ref, t_ref, store_shift):
    h_out = out_ref.shape[0]
    n = src_ref.shape[2]
    wv = w_ref[...]
    cout = wv.shape[0]
    bias = jnp.broadcast_to(b_ref[...], (cout, n))
    scale = jnp.broadcast_to(s_ref[...], (cout, n // 2))
    shift = jnp.broadcast_to(t_ref[...], (cout, n // 2))
    p0 = src_ref[0]
    for hp in range(h_out):
        p1 = src_ref[2 * hp + 1]
        p2 = src_ref[2 * hp + 2]
        t0 = jnp.concatenate([p0, p1], axis=0)
        t1 = jnp.concatenate([p1, p2], axis=0)
        y0 = jnp.dot(wv, t0, preferred_element_type=jnp.float32) + bias
        y1 = jnp.dot(wv, t1, preferred_element_type=jnp.float32) + bias
        pooled = _pool_cols(jnp.maximum(y0, 0.0) + jnp.maximum(y1, 0.0))
        z = pooled * scale + shift
        if store_shift:
            out_ref[hp] = jnp.concatenate([z, _shift1(z)], axis=0).astype(
                jnp.bfloat16)
        else:
            out_ref[hp] = z.astype(jnp.bfloat16)
        p0 = p2


def _enc_kernel(*args, n_conv, n_pw):
    x_ref = args[0]
    conv_refs = args[1:1 + 4 * n_conv]
    pw_refs = args[1 + 4 * n_conv:1 + 4 * (n_conv + n_pw)]
    o_ref = args[1 + 4 * (n_conv + n_pw)]
    bufs = args[2 + 4 * (n_conv + n_pw):]

    src = x_ref
    for k in range(n_conv):
        w_ref, b_ref, s_ref, t_ref = conv_refs[4 * k:4 * k + 4]
        if k <= 2:
            _conv_block_poly(src, bufs[k], w_ref, b_ref, s_ref, t_ref,
                             d_out=2 - k, bias_folded=(k <= 1),
                             split_sublanes=(k == 0))
        else:
            _conv_block(src, bufs[k], w_ref, b_ref, s_ref, t_ref,
                        store_shift=(k < n_conv - 1))
        src = bufs[k]

    v = src[0]
    u = None
    for k in range(n_pw):
        w_ref, b_ref, s_ref, t_ref = pw_refs[4 * k:4 * k + 4]
        cout = w_ref.shape[0]
        lanes = v.shape[1]
        u = jnp.maximum(
            jnp.dot(w_ref[...], v, preferred_element_type=jnp.float32)
            + jnp.broadcast_to(b_ref[...], (cout, lanes)), 0.0)
        u = u * jnp.broadcast_to(s_ref[...], (cout, lanes)) + jnp.broadcast_to(
            t_ref[...], (cout, lanes))
        v = u.astype(jnp.bfloat16)

    c_fin, lanes = u.shape
    g = lanes // 2
    sel = jnp.broadcast_to(
        jax.lax.iota(jnp.int32, g)[None, :] * 2, (c_fin, g))
    o_ref[...] = jnp.take_along_axis(u, sel, axis=1)


def _pack_conv_w(w, cin_store, eps, b, gm, bt, mn, vr, bias_col=None):
    kh, kw, cin, cout = w.shape
    cout_p = _rup(cout, 8)
    wt = jnp.zeros((cout_p, 4 * cin_store), jnp.float32)
    order = ((0, 0), (0, 1), (1, 0), (1, 1))
    blocks = {(0, 0): 0, (0, 1): 1, (1, 0): 2, (1, 1): 3}
    for (dy, dx) in order:
        blk = blocks[(dy, dx)]
        wt = wt.at[:cout, blk * cin_store:blk * cin_store + cin].set(
            w[dy, dx].astype(jnp.float32).T)
    if bias_col is not None:
        wt = wt.at[:cout, bias_col].set(b)
    s_bn = gm / jnp.sqrt(vr + eps)
    bias = jnp.zeros((cout_p, 1), jnp.float32).at[:cout, 0].set(b)
    scale = jnp.zeros((cout_p, 1), jnp.float32).at[:cout, 0].set(0.25 * s_bn)
    shift = jnp.zeros((cout_p, 1), jnp.float32).at[:cout, 0].set(
        bt - mn * s_bn)
    return wt.astype(jnp.bfloat16), bias, scale, shift


def _pack_pw(w, eps, b, gm, bt, mn, vr):
    cin, cout = w.shape
    cin_p, cout_p = _rup(cin, 8), _rup(cout, 8)
    wt = jnp.zeros((cout_p, cin_p), jnp.float32).at[:cout, :cin].set(
        w.astype(jnp.float32).T)
    s_bn = gm / jnp.sqrt(vr + eps)
    bias = jnp.zeros((cout_p, 1), jnp.float32).at[:cout, 0].set(b)
    scale = jnp.zeros((cout_p, 1), jnp.float32).at[:cout, 0].set(s_bn)
    shift = jnp.zeros((cout_p, 1), jnp.float32).at[:cout, 0].set(
        bt - mn * s_bn)
    return wt.astype(jnp.bfloat16), bias, scale, shift


def _encode(x_nchw, conv_params, pw_params, eps=1e-5):
    B, cin, H, W = x_nchw.shape
    g = _G
    ng = B // g
    cs0 = 4

    xb = x_nchw.astype(jnp.bfloat16)
    ones = jnp.ones((B, cs0 - cin, H, W), jnp.bfloat16)
    xp = jnp.concatenate([xb, ones], axis=1)
    xp = xp.reshape(ng, g, cs0, H, W).transpose(0, 3, 2, 1, 4)
    xp = xp.reshape(ng, H, cs0, g * W)
    perm = sorted(range(g * W),
                  key=lambda p: ((p % W) & 1, (p % W >> 1) & 1,
                                 (p % W >> 2) & 1, p // W, p % W >> 3))
    hw = g * W // 2
    xa = jnp.take(xp, jnp.asarray(perm, jnp.int32), axis=3)
    xp = jnp.concatenate([xa[..., :hw], xa[..., hw:]], axis=2)

    weight_arrays = []
    h, w_, cin_store = H, W, cs0
    scratch = []
    n_conv = len(conv_params)
    for k, (wt, b, gm, bt, mn, vr) in enumerate(conv_params):
        bias_col = {0: cin, 1: 2 * cs0 - 1}.get(k)
        packed = _pack_conv_w(wt, cin_store, eps, b, gm, bt, mn, vr,
                              bias_col=bias_col)
        wt_p, b_p, s_p, t_p = packed
        if k == 0:
            s_p = s_p.at[2 * cs0 - 1, 0].set(0.0)
            t_p = t_p.at[2 * cs0 - 1, 0].set(1.0)
        weight_arrays += [wt_p, b_p, s_p, t_p]
        cout_p = wt_p.shape[0]
        h, w_ = (h - 1) // 2, w_ // 2
        store_c = cout_p if k == n_conv - 1 else 2 * cout_p
        scratch.append(pltpu.VMEM((h, store_c, g * w_), jnp.bfloat16))
        cin_store = cout_p

    c_fin = cin_store
    for (wt, b, gm, bt, mn, vr) in pw_params:
        packed = _pack_pw(wt, eps, b, gm, bt, mn, vr)
        weight_arrays += list(packed)
        c_fin = packed[0].shape[0]

    def _const_spec(arr):
        nd = arr.ndim
        return pl.BlockSpec(arr.shape, lambda bi, _nd=nd: (0,) * _nd)

    in_specs = [pl.BlockSpec((None, H, 2 * cs0, g * W // 2),
                             lambda bi: (bi, 0, 0, 0))]
    in_specs += [_const_spec(a) for a in weight_arrays]

    out = pl.pallas_call(
        functools.partial(_enc_kernel, n_conv=n_conv, n_pw=len(pw_params)),
        out_shape=jax.ShapeDtypeStruct((ng, c_fin, g), jnp.float32),
        grid=(ng,),
        in_specs=in_specs,
        out_specs=pl.BlockSpec((None, c_fin, g), lambda bi: (bi, 0, 0)),
        scratch_shapes=scratch,
        compiler_params=pltpu.CompilerParams(
            dimension_semantics=("parallel",)),
    )(xp, *weight_arrays)
    return out.transpose(0, 2, 1).reshape(B, c_fin, 1, 1)


def kernel(
    x_nchw,
    conv0_w, conv0_b, conv0_g, conv0_be, conv0_m, conv0_v,
    conv1_w, conv1_b, conv1_g, conv1_be, conv1_m, conv1_v,
    conv2_w, conv2_b, conv2_g, conv2_be, conv2_m, conv2_v,
    conv3_w, conv3_b, conv3_g, conv3_be, conv3_m, conv3_v,
    conv4_w, conv4_b, conv4_g, conv4_be, conv4_m, conv4_v,
    conv5_w, conv5_b, conv5_g, conv5_be, conv5_m, conv5_v,
    pw0_w, pw0_b, pw0_g, pw0_be, pw0_m, pw0_v,
    pw1_w, pw1_b, pw1_g, pw1_be, pw1_m, pw1_v,
):
    conv_params = [
        (conv0_w, conv0_b, conv0_g, conv0_be, conv0_m, conv0_v),
        (conv1_w, conv1_b, conv1_g, conv1_be, conv1_m, conv1_v),
        (conv2_w, conv2_b, conv2_g, conv2_be, conv2_m, conv2_v),
        (conv3_w, conv3_b, conv3_g, conv3_be, conv3_m, conv3_v),
        (conv4_w, conv4_b, conv4_g, conv4_be, conv4_m, conv4_v),
        (conv5_w, conv5_b, conv5_g, conv5_be, conv5_m, conv5_v),
    ]
    pw_params = [
        (pw0_w, pw0_b, pw0_g, pw0_be, pw0_m, pw0_v),
        (pw1_w, pw1_b, pw1_g, pw1_be, pw1_m, pw1_v),
    ]
    return _encode(x_nchw, conv_params, pw_params)
